# baseline (speedup 1.0000x reference)
import functools

import jax
import jax.numpy as jnp
from jax import lax
from jax.experimental import pallas as pl
from jax.experimental.pallas import tpu as pltpu
from jax.experimental.pallas import tpu_sc as plsc

EMBED_DIM = 64
SEQ_LEN = 200
NUM_WORKERS = 32


@functools.lru_cache(maxsize=None)
def _make_sc_kernel(total_rows: int):
    rows_per_w = total_rows // NUM_WORKERS
    n_chunks = rows_per_w // SEQ_LEN
    mesh = plsc.VectorSubcoreMesh(core_axis_name="c", subcore_axis_name="s")

    @functools.partial(
        pl.kernel,
        mesh=mesh,
        compiler_params=pltpu.CompilerParams(use_tc_tiling_on_sc=False),
        out_type=jax.ShapeDtypeStruct((total_rows, EMBED_DIM), jnp.float32),
        scratch_types=[
            pltpu.VMEM((SEQ_LEN,), jnp.int32),
            pltpu.VMEM((SEQ_LEN, EMBED_DIM), jnp.float32),
            pltpu.VMEM((SEQ_LEN, EMBED_DIM), jnp.float32),
            pltpu.SemaphoreType.DMA,
        ],
    )
    def k(ids_hbm, tok_hbm, pos_hbm, out_hbm, idx_v, buf_v, pos_v, sem):
        wid = lax.axis_index("s") * 2 + lax.axis_index("c")
        pltpu.sync_copy(pos_hbm, pos_v)

        def chunk_body(g, carry):
            base = wid * rows_per_w + g * SEQ_LEN
            pltpu.sync_copy(ids_hbm.at[pl.ds(base, SEQ_LEN)], idx_v)
            pltpu.async_copy(tok_hbm.at[idx_v], buf_v, sem).wait()

            def add_body(r, c2):
                for c in range(EMBED_DIM // 16):
                    sl = pl.ds(c * 16, 16)
                    buf_v[r, sl] = buf_v[r, sl] + pos_v[r, sl]
                return c2

            lax.fori_loop(0, SEQ_LEN, add_body, None)
            pltpu.sync_copy(buf_v, out_hbm.at[pl.ds(base, SEQ_LEN)])
            return carry

        lax.fori_loop(0, n_chunks, chunk_body, None)

    return k


@jax.jit
def kernel(input_ids, token_table, position_table):
    batch, seq = input_ids.shape
    ids_flat = input_ids.reshape(-1).astype(jnp.int32)
    pos = position_table[:seq].astype(jnp.float32)
    out = _make_sc_kernel(batch * seq)(ids_flat, token_table, pos)
    return out.reshape(batch, seq, EMBED_DIM)


# speedup vs baseline: 1.1577x; 1.1577x over previous
import functools

import jax
import jax.numpy as jnp
from jax import lax
from jax.experimental import pallas as pl
from jax.experimental.pallas import tpu as pltpu
from jax.experimental.pallas import tpu_sc as plsc

EMBED_DIM = 64
SEQ_LEN = 200
NUM_WORKERS = 32
NBUF = 4


@functools.lru_cache(maxsize=None)
def _make_sc_kernel(total_rows: int):
    rows_per_w = total_rows // NUM_WORKERS
    n_chunks = rows_per_w // SEQ_LEN
    n_groups = n_chunks // NBUF
    mesh = plsc.VectorSubcoreMesh(core_axis_name="c", subcore_axis_name="s")

    @functools.partial(
        pl.kernel,
        mesh=mesh,
        compiler_params=pltpu.CompilerParams(use_tc_tiling_on_sc=False),
        out_type=jax.ShapeDtypeStruct((total_rows, EMBED_DIM), jnp.float32),
        scratch_types=[
            pltpu.VMEM((rows_per_w,), jnp.int32),
            pltpu.VMEM((SEQ_LEN, EMBED_DIM), jnp.float32),
        ]
        + [pltpu.VMEM((SEQ_LEN, EMBED_DIM), jnp.float32) for _ in range(NBUF)]
        + [pltpu.SemaphoreType.DMA for _ in range(2 * NBUF)],
    )
    def k(ids_hbm, tok_hbm, pos_hbm, out_hbm, idx_v, pos_v, *rest):
        bufs = rest[:NBUF]
        sg = rest[NBUF : 2 * NBUF]
        so = rest[2 * NBUF : 3 * NBUF]
        wid = lax.axis_index("s") * 2 + lax.axis_index("c")
        row0 = wid * rows_per_w
        pltpu.sync_copy(pos_hbm, pos_v)
        pltpu.sync_copy(ids_hbm.at[pl.ds(row0, rows_per_w)], idx_v)

        def gather_start(b, g):
            pltpu.async_copy(
                tok_hbm.at[idx_v.at[pl.ds(g * SEQ_LEN, SEQ_LEN)]], bufs[b], sg[b]
            )

        def gather_wait(b, g):
            pltpu.make_async_copy(
                tok_hbm.at[idx_v.at[pl.ds(g * SEQ_LEN, SEQ_LEN)]], bufs[b], sg[b]
            ).wait()

        def out_start(b, g):
            pltpu.async_copy(bufs[b], out_hbm.at[pl.ds(row0 + g * SEQ_LEN, SEQ_LEN)], so[b])

        def out_wait(b, g):
            pltpu.make_async_copy(
                bufs[b], out_hbm.at[pl.ds(row0 + g * SEQ_LEN, SEQ_LEN)], so[b]
            ).wait()

        for b in range(NBUF):
            gather_start(b, b)

        def group_body(i, carry):
            go = i * NBUF
            for b in range(NBUF):
                g = go + b
                gather_wait(b, g)

                def add_body(r, c2, b=b):
                    for rr in range(2):
                        row = r * 2 + rr
                        for c in range(EMBED_DIM // 16):
                            sl = pl.ds(c * 16, 16)
                            plsc.addupdate(bufs[b].at[row, sl], pos_v[row, sl])
                    return c2

                lax.fori_loop(0, SEQ_LEN // 2, add_body, None)
                out_start(b, g)

            @pl.when(i < n_groups - 1)
            def _():
                for b in range(NBUF):
                    g = go + b
                    out_wait(b, g)
                    gather_start(b, g + NBUF)

            return carry

        lax.fori_loop(0, n_groups, group_body, None)
        for b in range(NBUF):
            out_wait(b, n_chunks - NBUF + b)

    return k


@jax.jit
def kernel(input_ids, token_table, position_table):
    batch, seq = input_ids.shape
    ids_flat = input_ids.reshape(-1).astype(jnp.int32)
    pos = position_table[:seq].astype(jnp.float32)
    out = _make_sc_kernel(batch * seq)(ids_flat, token_table, pos)
    return out.reshape(batch, seq, EMBED_DIM)
